# Initial kernel scaffold; baseline (speedup 1.0000x reference)
import jax
import jax.numpy as jnp
from jax.experimental import pallas as pl


def kernel(x, edge_index, W1, b1, W2, b2, W3, b3):
    raise NotImplementedError("write your pallas kernel here")


# speedup vs baseline: 7.6695x; 7.6695x over previous
import functools

import jax
import jax.numpy as jnp
from jax import lax
from jax.experimental import pallas as pl
from jax.experimental.pallas import tpu as pltpu
from jax.experimental.pallas import tpu_sc as plsc

NC = 2
NS = 16
NW = NC * NS
K = 64
ZR = 64


def _cdiv(a, b):
    return (a + b - 1) // b


def _make_count(N, CW):
    nacc = NS * ZR * _cdiv(N + 1, NS * ZR)
    rps = nacc // NS

    mesh = plsc.VectorSubcoreMesh(core_axis_name="c", subcore_axis_name="s")

    @functools.partial(
        pl.kernel,
        out_type=jax.ShapeDtypeStruct((NC, nacc, 16), jnp.float32),
        mesh=mesh,
        scratch_types=[
            pltpu.VMEM((CW, K), jnp.int32),
            pltpu.VMEM((ZR, 16), jnp.float32),
            pltpu.VMEM((K, 16), jnp.float32),
            pltpu.VMEM_SHARED((nacc, 16), jnp.float32),
        ],
    )
    def count(dst_hbm, z_hbm, ones_hbm, out_hbm, idx_d, zbuf, ones, acc):
        c = lax.axis_index("c")
        s = lax.axis_index("s")
        wid = s * NC + c

        pltpu.sync_copy(z_hbm, zbuf)
        pltpu.sync_copy(ones_hbm, ones)
        for t in range(rps // ZR):
            pltpu.sync_copy(zbuf, acc.at[pl.ds(s * rps + t * ZR, ZR)])
        plsc.subcore_barrier()

        pltpu.sync_copy(dst_hbm.at[wid], idx_d)

        def body(k, carry):
            pltpu.sync_copy(ones, acc.at[idx_d.at[k]], add=True)
            return carry

        lax.fori_loop(0, CW, body, 0)
        plsc.subcore_barrier()

        pltpu.sync_copy(acc.at[pl.ds(s * rps, rps)],
                        out_hbm.at[c, pl.ds(s * rps, rps)])

    return count


def _make_prop(N, CW, D):
    nacc = NS * ZR * _cdiv(N + 1, NS * ZR)
    rps = nacc // NS

    mesh = plsc.VectorSubcoreMesh(core_axis_name="c", subcore_axis_name="s")

    @functools.partial(
        pl.kernel,
        out_type=jax.ShapeDtypeStruct((NC, nacc, D), jnp.float32),
        mesh=mesh,
        scratch_types=[
            pltpu.VMEM((CW // 2, K), jnp.int32),
            pltpu.VMEM((CW // 2, K), jnp.int32),
            pltpu.VMEM((2, K, D), jnp.float32),
            pltpu.VMEM_SHARED((nacc, D), jnp.float32),
            pltpu.SemaphoreType.DMA,
            pltpu.SemaphoreType.DMA,
        ],
    )
    def prop(y_hbm, src_hbm, dst_hbm, z_hbm, out_hbm,
             idx_s, idx_d, rows, acc, sem0, sem1):
        c = lax.axis_index("c")
        s = lax.axis_index("s")
        wid = s * NC + c

        pltpu.sync_copy(z_hbm, rows.at[0])
        for t in range(rps // K):
            pltpu.sync_copy(rows.at[0], acc.at[pl.ds(s * rps + t * K, K)])
        plsc.subcore_barrier()

        PH = CW // 2
        for phase in range(2):
            pltpu.sync_copy(src_hbm.at[wid, pl.ds(phase * PH, PH)], idx_s)
            pltpu.sync_copy(dst_hbm.at[wid, pl.ds(phase * PH, PH)], idx_d)
            pltpu.async_copy(y_hbm.at[idx_s.at[0]], rows.at[0], sem0)

            def body(i, carry):
                k0 = 2 * i
                pltpu.async_copy(y_hbm.at[idx_s.at[k0 + 1]], rows.at[1], sem1)
                pltpu.make_async_copy(y_hbm.at[idx_s.at[k0]], rows.at[0],
                                      sem0).wait()
                pltpu.sync_copy(rows.at[0], acc.at[idx_d.at[k0]], add=True)

                @pl.when(k0 + 2 < PH)
                def _():
                    pltpu.async_copy(y_hbm.at[idx_s.at[k0 + 2]], rows.at[0],
                                     sem0)

                pltpu.make_async_copy(y_hbm.at[idx_s.at[k0 + 1]], rows.at[1],
                                      sem1).wait()
                pltpu.sync_copy(rows.at[1], acc.at[idx_d.at[k0 + 1]], add=True)
                return carry

            lax.fori_loop(0, PH // 2, body, 0)
        plsc.subcore_barrier()

        pltpu.sync_copy(acc.at[pl.ds(s * rps, rps)],
                        out_hbm.at[c, pl.ds(s * rps, rps)])

    return prop


def _dis(c_ref):
    cnt = c_ref[0, :, 0:1] + c_ref[1, :, 0:1] + 1.0
    return lax.rsqrt(cnt)


def _tc_first(x_ref, w_ref, c_ref, o_ref):
    d = _dis(c_ref)
    o_ref[...] = jnp.dot(x_ref[...], w_ref[...],
                         preferred_element_type=jnp.float32) * d


def _tc_mid(p_ref, y_ref, c_ref, w_ref, b_ref, o_ref):
    d = _dis(c_ref)
    h = (p_ref[0] + p_ref[1] + y_ref[...]) * d + b_ref[...]
    h = jnp.maximum(h, 0.0)
    o_ref[...] = jnp.dot(h, w_ref[...], preferred_element_type=jnp.float32) * d


def _tc_pre3(p_ref, y_ref, c_ref, b_ref, o_ref):
    d = _dis(c_ref)
    h = (p_ref[0] + p_ref[1] + y_ref[...]) * d + b_ref[...]
    o_ref[...] = jnp.maximum(h, 0.0) * d


def _tc_last(q_ref, g_ref, c_ref, w_ref, b_ref, o_ref):
    d = _dis(c_ref)
    hh = (q_ref[0] + q_ref[1] + g_ref[...]) * d
    z = jnp.dot(hh, w_ref[...], preferred_element_type=jnp.float32) + b_ref[...]
    m = jnp.max(z, axis=-1, keepdims=True)
    e = jnp.exp(z - m)
    lse = jnp.log(jnp.sum(e, axis=-1, keepdims=True))
    o_ref[...] = z - m - lse


def _row_call(body, N, B, out_d, in_specs):
    return pl.pallas_call(
        body,
        grid=(N // B,),
        in_specs=in_specs,
        out_specs=pl.BlockSpec((B, out_d), lambda i: (i, 0)),
        out_shape=jax.ShapeDtypeStruct((N, out_d), jnp.float32),
    )


def kernel(x, edge_index, W1, b1, W2, b2, W3, b3):
    N, D_in = x.shape
    D_hid = W1.shape[1]
    D_out = W3.shape[1]
    E = edge_index.shape[1]

    CW = 4 * _cdiv(E, NW * K * 4)
    Epad = NW * CW * K

    src = edge_index[0].astype(jnp.int32)
    dst = edge_index[1].astype(jnp.int32)
    src3 = jnp.concatenate(
        [src, jnp.zeros((Epad - E,), jnp.int32)]).reshape(NW, CW, K)
    dst3 = jnp.concatenate(
        [dst, jnp.full((Epad - E,), N, jnp.int32)]).reshape(NW, CW, K)

    z16 = jnp.zeros((ZR, 16), jnp.float32)
    ones16 = jnp.ones((K, 16), jnp.float32)
    zD = jnp.zeros((K, D_hid), jnp.float32)

    cnt = _make_count(N, CW)(dst3, z16, ones16)
    prop = _make_prop(N, CW, D_hid)

    B = 1000
    spec_rows = pl.BlockSpec((B, D_hid), lambda i: (i, 0))
    spec_p = pl.BlockSpec((NC, B, D_hid), lambda i: (0, i, 0))
    spec_c = pl.BlockSpec((NC, B, 16), lambda i: (0, i, 0))
    spec_w = pl.BlockSpec((D_hid, D_hid), lambda i: (0, 0))
    spec_b = pl.BlockSpec((1, D_hid), lambda i: (0, 0))

    b1r, b2r = b1.reshape(1, -1), b2.reshape(1, -1)

    y1 = _row_call(_tc_first, N, B, D_hid,
                   [spec_rows, spec_w, spec_c])(x, W1, cnt)
    P1 = prop(y1, src3, dst3, zD)
    y2 = _row_call(_tc_mid, N, B, D_hid,
                   [spec_p, spec_rows, spec_c, spec_w, spec_b])(
                       P1, y1, cnt, W2, b1r)
    P2 = prop(y2, src3, dst3, zD)
    g = _row_call(_tc_pre3, N, B, D_hid,
                  [spec_p, spec_rows, spec_c, spec_b])(P2, y2, cnt, b2r)
    Q = prop(g, src3, dst3, zD)

    spec_w3 = pl.BlockSpec((D_hid, D_out), lambda i: (0, 0))
    spec_b3 = pl.BlockSpec((1, D_out), lambda i: (0, 0))
    out = _row_call(_tc_last, N, B, D_out,
                    [spec_p, spec_rows, spec_c, spec_w3, spec_b3])(
                        Q, g, cnt, W3, b3.reshape(1, -1))
    return out
